# baseline (speedup 1.0000x reference)
import jax
import jax.numpy as jnp
from jax import lax
from jax.experimental import pallas as pl
from jax.experimental.pallas import tpu as pltpu

BN_EPS = 1e-5
_SUB = 8


@jax.jit
def _forward(x_nchw, w_oihw, gamma, beta):
    N, Cin, H, W = x_nchw.shape
    Cout = w_oihw.shape[0]
    assert w_oihw.shape[1:] == (Cin, 3, 3)
    assert H % 2 == 0 and W % 2 == 0 and W % _SUB == 0
    Ho, Wo = H // 2, W // 2
    HW = H * W
    HWo = Ho * Wo
    Cp = ((Cout + 127) // 128) * 128
    G = 2 * Cin
    K2 = 6 * G
    NH = 2 if (N % 2 == 0 and N >= 2) else 1
    n_per = N // NH
    vmem_limit = 96 * 1024 * 1024

    xr = x_nchw.reshape(N, Cin, HW)

    wt = jnp.transpose(w_oihw.astype(jnp.float32), (2, 3, 1, 0))
    wt = jnp.pad(wt, ((0, 1), (0, 0), (0, 0), (0, Cp - Cout)))
    w2 = wt.reshape(2, 2, 3, Cin, Cp).transpose(0, 2, 1, 3, 4)
    w2 = w2.reshape(K2, Cp).astype(jnp.bfloat16)

    gamma_p = jnp.pad(gamma.astype(jnp.float32), (0, Cp - Cout))
    beta_p = jnp.pad(beta.astype(jnp.float32), (0, Cp - Cout))

    def stats_kernel(x_ref, w_ref, sum_ref, sq_ref, ymax_ref, ymin_ref,
                     xq_ref, col_ref, rmax_ref, rmin_ref):
        @pl.when(pl.program_id(1) == 0)
        def _init():
            sum_ref[...] = jnp.zeros_like(sum_ref)
            sq_ref[...] = jnp.zeros_like(sq_ref)
            xq_ref[...] = jnp.zeros_like(xq_ref)

        xb = jnp.transpose(x_ref[...].astype(jnp.bfloat16))
        xb = xb.reshape(H, W, Cin)
        xq_ref[1:H + 1, 1:W + 1, 0:Cin] = xb
        xq_ref[0:H, 1:W + 1, Cin:G] = xb
        for j2 in range(2):
            for kw in range(3):
                j = j2 * 3 + kw
                blk = xq_ref[j2 * 2:j2 * 2 + H, kw:kw + W, :].reshape(HW, G)
                col_ref[:, j * G:(j + 1) * G] = blk
        y = jnp.dot(col_ref[...], w_ref[...],
                    preferred_element_type=jnp.float32)
        y3 = y.reshape(HW // _SUB, _SUB, Cp)
        sum_ref[...] += jnp.sum(y3, axis=0)
        sq_ref[...] += jnp.sum(y3 * y3, axis=0)
        y4 = y.reshape(Ho, 2, W, Cp)
        rmax_ref[...] = jnp.maximum(y4[:, 0], y4[:, 1])
        rmin_ref[...] = jnp.minimum(y4[:, 0], y4[:, 1])
        ymax_ref[...] = jnp.maximum(
            rmax_ref[:, pl.ds(0, Wo, stride=2), :],
            rmax_ref[:, pl.ds(1, Wo, stride=2), :]).astype(jnp.bfloat16)
        ymin_ref[...] = jnp.minimum(
            rmin_ref[:, pl.ds(0, Wo, stride=2), :],
            rmin_ref[:, pl.ds(1, Wo, stride=2), :]).astype(jnp.bfloat16)

    conv_flops = 2 * N * HW * K2 * Cp
    sums_nh, sqs_nh, ymax, ymin = pl.pallas_call(
        stats_kernel,
        out_shape=(jax.ShapeDtypeStruct((NH, _SUB, Cp), jnp.float32),
                   jax.ShapeDtypeStruct((NH, _SUB, Cp), jnp.float32),
                   jax.ShapeDtypeStruct((N, Ho, Wo, Cp), jnp.bfloat16),
                   jax.ShapeDtypeStruct((N, Ho, Wo, Cp), jnp.bfloat16)),
        grid=(NH, n_per),
        in_specs=[
            pl.BlockSpec((None, Cin, HW),
                         lambda h, m: (h * n_per + m, 0, 0)),
            pl.BlockSpec((K2, Cp), lambda h, m: (0, 0)),
        ],
        out_specs=(pl.BlockSpec((None, _SUB, Cp), lambda h, m: (h, 0, 0)),
                   pl.BlockSpec((None, _SUB, Cp), lambda h, m: (h, 0, 0)),
                   pl.BlockSpec((None, Ho, Wo, Cp),
                                lambda h, m: (h * n_per + m, 0, 0, 0)),
                   pl.BlockSpec((None, Ho, Wo, Cp),
                                lambda h, m: (h * n_per + m, 0, 0, 0))),
        scratch_shapes=[pltpu.VMEM((H + 2, W + 2, G), jnp.bfloat16),
                        pltpu.VMEM((HW, K2), jnp.bfloat16),
                        pltpu.VMEM((Ho, W, Cp), jnp.float32),
                        pltpu.VMEM((Ho, W, Cp), jnp.float32)],
        compiler_params=pltpu.CompilerParams(
            dimension_semantics=("parallel", "arbitrary"),
            vmem_limit_bytes=vmem_limit),
        cost_estimate=pl.CostEstimate(
            flops=conv_flops, transcendentals=0,
            bytes_accessed=xr.size * 4 + w2.size * 2
            + 2 * N * Ho * Wo * Cp * 2),
    )(xr, w2)

    cnt = jnp.float32(N * H * W)
    sums = jnp.sum(sums_nh, axis=(0, 1))
    sqs = jnp.sum(sqs_nh, axis=(0, 1))
    mean = sums / cnt
    var = jnp.maximum(sqs / cnt - mean * mean, 0.0)
    scale = (gamma_p * lax.rsqrt(var + BN_EPS)).reshape(1, Cp)
    shift = (beta_p - mean * gamma_p * lax.rsqrt(var + BN_EPS)).reshape(1, Cp)

    def apply_kernel(mx_ref, mn_ref, sc_ref, sh_ref, o_ref):
        sc = sc_ref[...]
        sh = sh_ref[...]
        a = mx_ref[...].astype(jnp.float32) * sc + sh
        b = mn_ref[...].astype(jnp.float32) * sc + sh
        y = jnp.maximum(jnp.maximum(a, b), 0.0)
        o_ref[...] = jnp.transpose(y.reshape(HWo, Cp))

    out_p = pl.pallas_call(
        apply_kernel,
        out_shape=jax.ShapeDtypeStruct((N, Cp, HWo), jnp.float32),
        grid=(N,),
        in_specs=[
            pl.BlockSpec((None, Ho, Wo, Cp), lambda n: (n, 0, 0, 0)),
            pl.BlockSpec((None, Ho, Wo, Cp), lambda n: (n, 0, 0, 0)),
            pl.BlockSpec((1, Cp), lambda n: (0, 0)),
            pl.BlockSpec((1, Cp), lambda n: (0, 0)),
        ],
        out_specs=pl.BlockSpec((None, Cp, HWo), lambda n: (n, 0, 0)),
        compiler_params=pltpu.CompilerParams(
            dimension_semantics=("parallel",),
            vmem_limit_bytes=vmem_limit),
        cost_estimate=pl.CostEstimate(
            flops=4 * N * HWo * Cp, transcendentals=0,
            bytes_accessed=2 * N * HWo * Cp * 2 + N * HWo * Cp * 4),
    )(ymax, ymin, scale, shift)

    out = out_p[:, :Cout, :] if Cout != Cp else out_p
    return out.reshape(N, Cout, Ho, Wo)


def kernel(x_nchw, w_oihw, gamma, beta):
    return _forward(x_nchw, w_oihw, gamma, beta)


# speedup vs baseline: 1.0190x; 1.0190x over previous
import jax
import jax.numpy as jnp
from jax import lax
from jax.experimental import pallas as pl
from jax.experimental.pallas import tpu as pltpu

BN_EPS = 1e-5
_SUB = 8


@jax.jit
def _forward(x_nchw, w_oihw, gamma, beta):
    N, Cin, H, W = x_nchw.shape
    Cout = w_oihw.shape[0]
    assert w_oihw.shape[1:] == (Cin, 3, 3)
    assert H % 2 == 0 and W % 2 == 0 and W % _SUB == 0
    Ho, Wo = H // 2, W // 2
    HW = H * W
    HWo = Ho * Wo
    Cp = ((Cout + 127) // 128) * 128
    G = 2 * Cin
    K2 = 6 * G
    vmem_limit = 48 * 1024 * 1024

    wt = jnp.transpose(w_oihw.astype(jnp.float32), (2, 3, 1, 0))
    wt = jnp.pad(wt, ((0, 1), (0, 0), (0, 0), (0, Cp - Cout)))
    w2 = wt.reshape(2, 2, 3, Cin, Cp).transpose(0, 2, 1, 3, 4)
    w2 = w2.reshape(K2, Cp).astype(jnp.bfloat16)

    gamma_p = jnp.pad(gamma.astype(jnp.float32), (0, Cp - Cout))
    beta_p = jnp.pad(beta.astype(jnp.float32), (0, Cp - Cout))

    def stats_kernel(x_ref, w_ref, sum_ref, sq_ref, ymax_ref, ymin_ref,
                     xq_ref, col_ref, rmax_ref, rmin_ref):
        xq_ref[...] = jnp.zeros_like(xq_ref)
        xb = jnp.transpose(x_ref[...].astype(jnp.bfloat16), (1, 2, 0))
        xq_ref[1:H + 1, 1:W + 1, 0:Cin] = xb
        xq_ref[0:H, 1:W + 1, Cin:G] = xb
        for j2 in range(2):
            for kw in range(3):
                j = j2 * 3 + kw
                blk = xq_ref[j2 * 2:j2 * 2 + H, kw:kw + W, :].reshape(HW, G)
                col_ref[:, j * G:(j + 1) * G] = blk
        y = jnp.dot(col_ref[...], w_ref[...],
                    preferred_element_type=jnp.float32)
        y3 = y.reshape(HW // _SUB, _SUB, Cp)
        sum_ref[...] = jnp.sum(y3, axis=0)
        sq_ref[...] = jnp.sum(y3 * y3, axis=0)
        y4 = y.reshape(Ho, 2, W, Cp)
        rmax_ref[...] = jnp.maximum(y4[:, 0], y4[:, 1])
        rmin_ref[...] = jnp.minimum(y4[:, 0], y4[:, 1])
        ymax_ref[...] = jnp.maximum(
            rmax_ref[:, pl.ds(0, Wo, stride=2), :],
            rmax_ref[:, pl.ds(1, Wo, stride=2), :]).astype(jnp.bfloat16)
        ymin_ref[...] = jnp.minimum(
            rmin_ref[:, pl.ds(0, Wo, stride=2), :],
            rmin_ref[:, pl.ds(1, Wo, stride=2), :]).astype(jnp.bfloat16)

    conv_flops = 2 * N * HW * K2 * Cp
    sums_n, sqs_n, ymax, ymin = pl.pallas_call(
        stats_kernel,
        out_shape=(jax.ShapeDtypeStruct((N, _SUB, Cp), jnp.float32),
                   jax.ShapeDtypeStruct((N, _SUB, Cp), jnp.float32),
                   jax.ShapeDtypeStruct((N, Ho, Wo, Cp), jnp.bfloat16),
                   jax.ShapeDtypeStruct((N, Ho, Wo, Cp), jnp.bfloat16)),
        grid=(N,),
        in_specs=[
            pl.BlockSpec((None, Cin, H, W), lambda n: (n, 0, 0, 0)),
            pl.BlockSpec((K2, Cp), lambda n: (0, 0)),
        ],
        out_specs=(pl.BlockSpec((None, _SUB, Cp), lambda n: (n, 0, 0)),
                   pl.BlockSpec((None, _SUB, Cp), lambda n: (n, 0, 0)),
                   pl.BlockSpec((None, Ho, Wo, Cp), lambda n: (n, 0, 0, 0)),
                   pl.BlockSpec((None, Ho, Wo, Cp), lambda n: (n, 0, 0, 0))),
        scratch_shapes=[pltpu.VMEM((H + 2, W + 2, G), jnp.bfloat16),
                        pltpu.VMEM((HW, K2), jnp.bfloat16),
                        pltpu.VMEM((Ho, W, Cp), jnp.float32),
                        pltpu.VMEM((Ho, W, Cp), jnp.float32)],
        compiler_params=pltpu.CompilerParams(
            dimension_semantics=("parallel",),
            vmem_limit_bytes=vmem_limit),
        cost_estimate=pl.CostEstimate(
            flops=conv_flops, transcendentals=0,
            bytes_accessed=x_nchw.size * 4 + w2.size * 2
            + 2 * N * Ho * Wo * Cp * 2),
    )(x_nchw, w2)

    cnt = jnp.float32(N * H * W)
    sums = jnp.sum(sums_n, axis=(0, 1))
    sqs = jnp.sum(sqs_n, axis=(0, 1))
    mean = sums / cnt
    var = jnp.maximum(sqs / cnt - mean * mean, 0.0)
    scale = (gamma_p * lax.rsqrt(var + BN_EPS)).reshape(1, Cp)
    shift = (beta_p - mean * gamma_p * lax.rsqrt(var + BN_EPS)).reshape(1, Cp)

    def apply_kernel(mx_ref, mn_ref, sc_ref, sh_ref, o_ref):
        sc = sc_ref[...]
        sh = sh_ref[...]
        a = mx_ref[...].astype(jnp.float32) * sc + sh
        b = mn_ref[...].astype(jnp.float32) * sc + sh
        y = jnp.maximum(jnp.maximum(a, b), 0.0)
        yt = jnp.transpose(y.reshape(HWo, Cp))
        o_ref[...] = yt.reshape(Cp, Ho, Wo)

    out_p = pl.pallas_call(
        apply_kernel,
        out_shape=jax.ShapeDtypeStruct((N, Cp, Ho, Wo), jnp.float32),
        grid=(N,),
        in_specs=[
            pl.BlockSpec((None, Ho, Wo, Cp), lambda n: (n, 0, 0, 0)),
            pl.BlockSpec((None, Ho, Wo, Cp), lambda n: (n, 0, 0, 0)),
            pl.BlockSpec((1, Cp), lambda n: (0, 0)),
            pl.BlockSpec((1, Cp), lambda n: (0, 0)),
        ],
        out_specs=pl.BlockSpec((None, Cp, Ho, Wo), lambda n: (n, 0, 0, 0)),
        compiler_params=pltpu.CompilerParams(
            dimension_semantics=("parallel",),
            vmem_limit_bytes=vmem_limit),
        cost_estimate=pl.CostEstimate(
            flops=4 * N * HWo * Cp, transcendentals=0,
            bytes_accessed=2 * N * HWo * Cp * 2 + N * HWo * Cp * 4),
    )(ymax, ymin, scale, shift)

    return out_p if Cout == Cp else out_p[:, :Cout]


def kernel(x_nchw, w_oihw, gamma, beta):
    return _forward(x_nchw, w_oihw, gamma, beta)


# speedup vs baseline: 1.3048x; 1.2805x over previous
import jax
import jax.numpy as jnp
from jax import lax
from jax.experimental import pallas as pl
from jax.experimental.pallas import tpu as pltpu

BN_EPS = 1e-5
_SUB = 8


@jax.jit
def _forward(x_nchw, w_oihw, gamma, beta):
    N, Cin, H, W = x_nchw.shape
    Cout = w_oihw.shape[0]
    assert w_oihw.shape[1:] == (Cin, 3, 3)
    assert H % 2 == 0 and W % 2 == 0 and W % _SUB == 0
    Ho, Wo = H // 2, W // 2
    HW = H * W
    HWo = Ho * Wo
    Cp = ((Cout + 127) // 128) * 128
    G = 2 * Cin
    K2 = 6 * G
    vmem_limit = 48 * 1024 * 1024

    wt = jnp.transpose(w_oihw.astype(jnp.float32), (2, 3, 1, 0))
    wt = jnp.pad(wt, ((0, 1), (0, 0), (0, 0), (0, Cp - Cout)))
    w2 = wt.reshape(2, 2, 3, Cin, Cp).transpose(0, 2, 1, 3, 4)
    w2 = w2.reshape(K2, Cp).astype(jnp.bfloat16)

    gamma_p = jnp.pad(gamma.astype(jnp.float32), (0, Cp - Cout))
    beta_p = jnp.pad(beta.astype(jnp.float32), (0, Cp - Cout))

    def stats_kernel(x_ref, w_ref, sum_ref, sq_ref, ymax_ref, ymin_ref,
                     xq_ref, col_ref, rmax_ref, rmin_ref):
        zrow = jnp.zeros((1, W + 2, G), jnp.bfloat16)
        zcol = jnp.zeros((H + 2, 1, G), jnp.bfloat16)
        xq_ref[0:1] = zrow
        xq_ref[H + 1:H + 2] = zrow
        xq_ref[:, 0:1, :] = zcol
        xq_ref[:, W + 1:W + 2, :] = zcol
        xb = jnp.transpose(x_ref[...].astype(jnp.bfloat16), (1, 2, 0))
        xq_ref[1:H + 1, 1:W + 1, 0:Cin] = xb
        xq_ref[0:H, 1:W + 1, Cin:G] = xb
        for j2 in range(2):
            for kw in range(3):
                j = j2 * 3 + kw
                blk = xq_ref[j2 * 2:j2 * 2 + H, kw:kw + W, :].reshape(HW, G)
                col_ref[:, j * G:(j + 1) * G] = blk
        y = jnp.dot(col_ref[...], w_ref[...],
                    preferred_element_type=jnp.float32)
        y3 = y.reshape(HW // _SUB, _SUB, Cp)
        sum_ref[...] = jnp.sum(y3, axis=0)
        sq_ref[...] = jnp.sum(y3 * y3, axis=0)
        y4 = y.reshape(Ho, 2, W, Cp)
        rmax_ref[...] = jnp.maximum(y4[:, 0], y4[:, 1])
        rmin_ref[...] = jnp.minimum(y4[:, 0], y4[:, 1])
        ymax_ref[...] = jnp.maximum(
            rmax_ref[:, pl.ds(0, Wo, stride=2), :],
            rmax_ref[:, pl.ds(1, Wo, stride=2), :]).astype(jnp.bfloat16)
        ymin_ref[...] = jnp.minimum(
            rmin_ref[:, pl.ds(0, Wo, stride=2), :],
            rmin_ref[:, pl.ds(1, Wo, stride=2), :]).astype(jnp.bfloat16)

    conv_flops = 2 * N * HW * K2 * Cp
    sums_n, sqs_n, ymax, ymin = pl.pallas_call(
        stats_kernel,
        out_shape=(jax.ShapeDtypeStruct((N, _SUB, Cp), jnp.float32),
                   jax.ShapeDtypeStruct((N, _SUB, Cp), jnp.float32),
                   jax.ShapeDtypeStruct((N, Ho, Wo, Cp), jnp.bfloat16),
                   jax.ShapeDtypeStruct((N, Ho, Wo, Cp), jnp.bfloat16)),
        grid=(N,),
        in_specs=[
            pl.BlockSpec((None, Cin, H, W), lambda n: (n, 0, 0, 0)),
            pl.BlockSpec((K2, Cp), lambda n: (0, 0)),
        ],
        out_specs=(pl.BlockSpec((None, _SUB, Cp), lambda n: (n, 0, 0)),
                   pl.BlockSpec((None, _SUB, Cp), lambda n: (n, 0, 0)),
                   pl.BlockSpec((None, Ho, Wo, Cp), lambda n: (n, 0, 0, 0)),
                   pl.BlockSpec((None, Ho, Wo, Cp), lambda n: (n, 0, 0, 0))),
        scratch_shapes=[pltpu.VMEM((H + 2, W + 2, G), jnp.bfloat16),
                        pltpu.VMEM((HW, K2), jnp.bfloat16),
                        pltpu.VMEM((Ho, W, Cp), jnp.float32),
                        pltpu.VMEM((Ho, W, Cp), jnp.float32)],
        compiler_params=pltpu.CompilerParams(
            dimension_semantics=("parallel",),
            vmem_limit_bytes=vmem_limit),
        cost_estimate=pl.CostEstimate(
            flops=conv_flops, transcendentals=0,
            bytes_accessed=x_nchw.size * 4 + w2.size * 2
            + 2 * N * Ho * Wo * Cp * 2),
    )(x_nchw, w2)

    cnt = jnp.float32(N * H * W)
    sums = jnp.sum(sums_n, axis=(0, 1))
    sqs = jnp.sum(sqs_n, axis=(0, 1))
    mean = sums / cnt
    var = jnp.maximum(sqs / cnt - mean * mean, 0.0)
    scale = (gamma_p * lax.rsqrt(var + BN_EPS)).reshape(1, Cp)
    shift = (beta_p - mean * gamma_p * lax.rsqrt(var + BN_EPS)).reshape(1, Cp)

    def apply_kernel(mx_ref, mn_ref, sc_ref, sh_ref, o_ref):
        sc = sc_ref[...]
        sh = sh_ref[...]
        a = mx_ref[...].astype(jnp.float32) * sc + sh
        b = mn_ref[...].astype(jnp.float32) * sc + sh
        y = jnp.maximum(jnp.maximum(a, b), 0.0)
        o_ref[...] = jnp.transpose(y.reshape(HWo, Cp))

    out_p = pl.pallas_call(
        apply_kernel,
        out_shape=jax.ShapeDtypeStruct((N, Cp, HWo), jnp.float32),
        grid=(N,),
        in_specs=[
            pl.BlockSpec((None, Ho, Wo, Cp), lambda n: (n, 0, 0, 0)),
            pl.BlockSpec((None, Ho, Wo, Cp), lambda n: (n, 0, 0, 0)),
            pl.BlockSpec((1, Cp), lambda n: (0, 0)),
            pl.BlockSpec((1, Cp), lambda n: (0, 0)),
        ],
        out_specs=pl.BlockSpec((None, Cp, HWo), lambda n: (n, 0, 0)),
        compiler_params=pltpu.CompilerParams(
            dimension_semantics=("parallel",),
            vmem_limit_bytes=vmem_limit),
        cost_estimate=pl.CostEstimate(
            flops=4 * N * HWo * Cp, transcendentals=0,
            bytes_accessed=2 * N * HWo * Cp * 2 + N * HWo * Cp * 4),
    )(ymax, ymin, scale, shift)

    out = out_p if Cout == Cp else out_p[:, :Cout, :]
    return out.reshape(N, Cout, Ho, Wo)


def kernel(x_nchw, w_oihw, gamma, beta):
    return _forward(x_nchw, w_oihw, gamma, beta)
